# Initial kernel scaffold; baseline (speedup 1.0000x reference)
import functools

import jax
import jax.numpy as jnp
from jax import lax
from jax.experimental import pallas as pl
from jax.experimental.pallas import tpu as pltpu
from jax.experimental.pallas import tpu_sc as plsc

H = 128
N_NODES = 10000

NC = 2
NS = 16
NW = NC * NS

CH = 128
NCH = 80
EPT = NCH * CH
E_PAD = NW * EPT

ZR = 626
N_ACC = ZR * NS
CPT = N_NODES // NS

TA = 2048
TB = 2000
NT = N_NODES // TB


def _msg_body(ef_ref, w_ref, b_ref, out_ref):
    x = ef_ref[...]
    y = jnp.dot(x, w_ref[...], preferred_element_type=jnp.float32) + b_ref[...]
    out_ref[...] = jnp.maximum(y, 0.0)


def _scatter_body(msg_hbm, idx_hbm, zeros_hbm, out_hbm,
                  idx_v, buf0, buf1, acc, sem0, sem1):
    c = lax.axis_index("c")
    s = lax.axis_index("s")
    wid = c * NS + s
    pltpu.sync_copy(zeros_hbm, acc.at[pl.ds(s * ZR, ZR)])
    pltpu.sync_copy(idx_hbm.at[wid], idx_v)
    plsc.subcore_barrier()

    base = wid * EPT
    pltpu.async_copy(msg_hbm.at[pl.ds(base, CH)], buf0, sem0)

    def step(k, carry):
        j0 = 2 * k
        pltpu.async_copy(msg_hbm.at[pl.ds(base + (j0 + 1) * CH, CH)], buf1, sem1)
        pltpu.make_async_copy(msg_hbm.at[pl.ds(base + j0 * CH, CH)], buf0, sem0).wait()
        pltpu.sync_copy(buf0, acc.at[idx_v.at[j0]], add=True)

        @pl.when(k < NCH // 2 - 1)
        def _():
            pltpu.async_copy(msg_hbm.at[pl.ds(base + (j0 + 2) * CH, CH)], buf0, sem0)

        pltpu.make_async_copy(msg_hbm.at[pl.ds(base + (j0 + 1) * CH, CH)], buf1, sem1).wait()
        pltpu.sync_copy(buf1, acc.at[idx_v.at[j0 + 1]], add=True)
        return carry

    lax.fori_loop(0, NCH // 2, step, 0)
    plsc.subcore_barrier()
    pltpu.sync_copy(acc.at[pl.ds(s * CPT, CPT)], out_hbm.at[c, pl.ds(s * CPT, CPT)])


def _gru_body(ef_ref, p0_ref, p1_ref, wih_ref, whh_ref, bih_ref, bhh_ref,
              out_ref, gi_scr):
    pid = pl.program_id(0)
    h = ef_ref[...]
    gh = jnp.dot(h, whh_ref[...], preferred_element_type=jnp.float32) + bhh_ref[...]

    @pl.when(pid < NT)
    def _():
        x = p0_ref[...] + p1_ref[...]
        gi_scr[...] = jnp.dot(x, wih_ref[...], preferred_element_type=jnp.float32)

    @pl.when(pid >= NT)
    def _():
        gi_scr[...] = jnp.zeros_like(gi_scr)

    gi = gi_scr[...] + bih_ref[...]
    r = jax.nn.sigmoid(gi[:, :H] + gh[:, :H])
    z = jax.nn.sigmoid(gi[:, H:2 * H] + gh[:, H:2 * H])
    n = jnp.tanh(gi[:, 2 * H:] + r * gh[:, 2 * H:])
    out_ref[...] = (1.0 - z) * n + z * h


def kernel(edge_features, edge_index, W_msg_w, W_msg_b, W_ih, W_hh, b_ih, b_hh):
    E = edge_features.shape[0]

    last_in_blk = (E - 1) // TA
    message = pl.pallas_call(
        _msg_body,
        grid=(E_PAD // TA,),
        in_specs=[
            pl.BlockSpec((TA, H), lambda i: (jnp.minimum(i, last_in_blk), 0)),
            pl.BlockSpec((H, H), lambda i: (0, 0)),
            pl.BlockSpec((1, H), lambda i: (0, 0)),
        ],
        out_specs=pl.BlockSpec((TA, H), lambda i: (i, 0)),
        out_shape=jax.ShapeDtypeStruct((E_PAD, H), jnp.float32),
    )(edge_features, W_msg_w.T, W_msg_b.reshape(1, H))

    idx = jnp.full((E_PAD,), N_NODES, dtype=jnp.int32).at[:E].set(edge_index[1])
    idx3 = idx.reshape(NW, NCH, CH)
    zeros = jnp.zeros((ZR, H), dtype=jnp.float32)

    mesh = plsc.VectorSubcoreMesh(core_axis_name="c", subcore_axis_name="s")
    scatter = pl.kernel(
        _scatter_body,
        out_type=jax.ShapeDtypeStruct((NC, N_NODES, H), jnp.float32),
        mesh=mesh,
        scratch_types=[
            pltpu.VMEM((NCH, CH), jnp.int32),
            pltpu.VMEM((CH, H), jnp.float32),
            pltpu.VMEM((CH, H), jnp.float32),
            pltpu.VMEM_SHARED((N_ACC, H), jnp.float32),
            pltpu.SemaphoreType.DMA,
            pltpu.SemaphoreType.DMA,
        ],
    )
    partials = scatter(message, idx3, zeros)

    out = pl.pallas_call(
        _gru_body,
        grid=(E // TB,),
        in_specs=[
            pl.BlockSpec((TB, H), lambda i: (i, 0)),
            pl.BlockSpec((TB, H), lambda i: (jnp.minimum(i, NT - 1), 0)),
            pl.BlockSpec((TB, H), lambda i: (jnp.minimum(i, NT - 1), 0)),
            pl.BlockSpec((H, 3 * H), lambda i: (0, 0)),
            pl.BlockSpec((H, 3 * H), lambda i: (0, 0)),
            pl.BlockSpec((1, 3 * H), lambda i: (0, 0)),
            pl.BlockSpec((1, 3 * H), lambda i: (0, 0)),
        ],
        out_specs=pl.BlockSpec((TB, H), lambda i: (i, 0)),
        out_shape=jax.ShapeDtypeStruct((E, H), jnp.float32),
        scratch_shapes=[pltpu.VMEM((TB, 3 * H), jnp.float32)],
    )(edge_features, partials[0], partials[1], W_ih.T, W_hh.T,
      b_ih.reshape(1, 3 * H), b_hh.reshape(1, 3 * H))
    return out


# speedup vs baseline: 3.1281x; 3.1281x over previous
import functools

import jax
import jax.numpy as jnp
from jax import lax
from jax.experimental import pallas as pl
from jax.experimental.pallas import tpu as pltpu
from jax.experimental.pallas import tpu_sc as plsc

H = 128
N_NODES = 10000

NC = 2
NS = 16
NW = NC * NS

CH = 128
NCH = 80
EPT = NCH * CH
E_PAD = NW * EPT

ZR = 640
N_ACC = ZR * NS
CPT = 632
N_OUT = CPT * NS

TA = 2048
TB = 2000
NT = N_NODES // TB


def _msg_body(ef_ref, w_ref, b_ref, out_ref):
    x = ef_ref[...]
    y = jnp.dot(x, w_ref[...], preferred_element_type=jnp.float32) + b_ref[...]
    out_ref[...] = jnp.maximum(y, 0.0)


def _scatter_body(msg_hbm, idx_hbm, zeros_hbm, out_hbm,
                  idx_v, buf0, buf1, acc, sem0, sem1):
    c = lax.axis_index("c")
    s = lax.axis_index("s")
    wid = c * NS + s
    pltpu.sync_copy(zeros_hbm, acc.at[pl.ds(s * ZR, ZR)])
    pltpu.sync_copy(idx_hbm.at[wid], idx_v)
    plsc.subcore_barrier()

    base = wid * EPT
    pltpu.async_copy(msg_hbm.at[pl.ds(base, CH)], buf0, sem0)

    def step(k, carry):
        j0 = 2 * k
        pltpu.async_copy(msg_hbm.at[pl.ds(base + (j0 + 1) * CH, CH)], buf1, sem1)
        pltpu.make_async_copy(msg_hbm.at[pl.ds(base + j0 * CH, CH)], buf0, sem0).wait()
        pltpu.sync_copy(buf0, acc.at[idx_v.at[j0]], add=True)

        @pl.when(k < NCH // 2 - 1)
        def _():
            pltpu.async_copy(msg_hbm.at[pl.ds(base + (j0 + 2) * CH, CH)], buf0, sem0)

        pltpu.make_async_copy(msg_hbm.at[pl.ds(base + (j0 + 1) * CH, CH)], buf1, sem1).wait()
        pltpu.sync_copy(buf1, acc.at[idx_v.at[j0 + 1]], add=True)
        return carry

    lax.fori_loop(0, NCH // 2, step, 0)
    plsc.subcore_barrier()
    pltpu.sync_copy(acc.at[pl.ds(s * CPT, CPT)], out_hbm.at[c, pl.ds(s * CPT, CPT)])


def _gru_body(ef_ref, p0_ref, p1_ref, wih_ref, whh_ref, bih_ref, bhh_ref,
              out_ref, gi_scr):
    pid = pl.program_id(0)
    h = ef_ref[...]
    gh = jnp.dot(h, whh_ref[...], preferred_element_type=jnp.float32) + bhh_ref[...]

    @pl.when(pid < NT)
    def _():
        x = p0_ref[...] + p1_ref[...]
        gi_scr[...] = jnp.dot(x, wih_ref[...], preferred_element_type=jnp.float32)

    @pl.when(pid >= NT)
    def _():
        gi_scr[...] = jnp.zeros_like(gi_scr)

    gi = gi_scr[...] + bih_ref[...]
    r = jax.nn.sigmoid(gi[:, :H] + gh[:, :H])
    z = jax.nn.sigmoid(gi[:, H:2 * H] + gh[:, H:2 * H])
    n = jnp.tanh(gi[:, 2 * H:] + r * gh[:, 2 * H:])
    out_ref[...] = (1.0 - z) * n + z * h


def kernel(edge_features, edge_index, W_msg_w, W_msg_b, W_ih, W_hh, b_ih, b_hh):
    E = edge_features.shape[0]

    last_in_blk = (E - 1) // TA
    message = pl.pallas_call(
        _msg_body,
        grid=(E_PAD // TA,),
        in_specs=[
            pl.BlockSpec((TA, H), lambda i: (jnp.minimum(i, last_in_blk), 0)),
            pl.BlockSpec((H, H), lambda i: (0, 0)),
            pl.BlockSpec((1, H), lambda i: (0, 0)),
        ],
        out_specs=pl.BlockSpec((TA, H), lambda i: (i, 0)),
        out_shape=jax.ShapeDtypeStruct((E_PAD, H), jnp.float32),
    )(edge_features, W_msg_w.T, W_msg_b.reshape(1, H))

    idx = jnp.full((E_PAD,), N_NODES, dtype=jnp.int32).at[:E].set(edge_index[1])
    idx3 = idx.reshape(NW, NCH, CH)
    zeros = jnp.zeros((ZR, H), dtype=jnp.float32)

    mesh = plsc.VectorSubcoreMesh(core_axis_name="c", subcore_axis_name="s")
    scatter = pl.kernel(
        _scatter_body,
        out_type=jax.ShapeDtypeStruct((NC, N_OUT, H), jnp.float32),
        mesh=mesh,
        scratch_types=[
            pltpu.VMEM((NCH, CH), jnp.int32),
            pltpu.VMEM((CH, H), jnp.float32),
            pltpu.VMEM((CH, H), jnp.float32),
            pltpu.VMEM_SHARED((N_ACC, H), jnp.float32),
            pltpu.SemaphoreType.DMA,
            pltpu.SemaphoreType.DMA,
        ],
    )
    partials = scatter(message, idx3, zeros)

    out = pl.pallas_call(
        _gru_body,
        grid=(E // TB,),
        in_specs=[
            pl.BlockSpec((TB, H), lambda i: (i, 0)),
            pl.BlockSpec((TB, H), lambda i: (jnp.minimum(i, NT - 1), 0)),
            pl.BlockSpec((TB, H), lambda i: (jnp.minimum(i, NT - 1), 0)),
            pl.BlockSpec((H, 3 * H), lambda i: (0, 0)),
            pl.BlockSpec((H, 3 * H), lambda i: (0, 0)),
            pl.BlockSpec((1, 3 * H), lambda i: (0, 0)),
            pl.BlockSpec((1, 3 * H), lambda i: (0, 0)),
        ],
        out_specs=pl.BlockSpec((TB, H), lambda i: (i, 0)),
        out_shape=jax.ShapeDtypeStruct((E, H), jnp.float32),
        scratch_shapes=[pltpu.VMEM((TB, 3 * H), jnp.float32)],
    )(edge_features, partials[0], partials[1], W_ih.T, W_hh.T,
      b_ih.reshape(1, 3 * H), b_hh.reshape(1, 3 * H))
    return out


# speedup vs baseline: 4.0343x; 1.2897x over previous
import functools

import jax
import jax.numpy as jnp
from jax import lax
from jax.experimental import pallas as pl
from jax.experimental.pallas import tpu as pltpu
from jax.experimental.pallas import tpu_sc as plsc

H = 128
N_NODES = 10000

NC = 2
NS = 16
NW = NC * NS

CH = 128
NCH = 80
EPT = NCH * CH
E_PAD = NW * EPT

ZR = 640
N_ACC = ZR * NS
CPT = 632
N_OUT = CPT * NS

TA = 2048
TB = 2000
NT = N_NODES // TB


def _msg_body(ef_ref, w_ref, b_ref, out_ref):
    x = ef_ref[...]
    y = jnp.dot(x, w_ref[...], preferred_element_type=jnp.float32) + b_ref[...]
    out_ref[...] = jnp.maximum(y, 0.0)


def _scatter_body(msg_hbm, idx_hbm, zeros_hbm, out_hbm,
                  idx_v, buf0, buf1, acc, sem0, sem1):
    c = lax.axis_index("c")
    s = lax.axis_index("s")
    wid = c * NS + s
    pltpu.sync_copy(zeros_hbm, acc.at[pl.ds(s * ZR, ZR)])
    pltpu.sync_copy(idx_hbm.at[wid], idx_v)
    plsc.subcore_barrier()

    base = wid * EPT
    pltpu.async_copy(msg_hbm.at[pl.ds(base, CH)], buf0, sem0)

    def step(k, carry):
        j0 = 2 * k
        pltpu.async_copy(msg_hbm.at[pl.ds(base + (j0 + 1) * CH, CH)], buf1, sem1)
        pltpu.make_async_copy(msg_hbm.at[pl.ds(base + j0 * CH, CH)], buf0, sem0).wait()
        pltpu.sync_copy(buf0, acc.at[idx_v.at[j0]], add=True)

        @pl.when(k < NCH // 2 - 1)
        def _():
            pltpu.async_copy(msg_hbm.at[pl.ds(base + (j0 + 2) * CH, CH)], buf0, sem0)

        pltpu.make_async_copy(msg_hbm.at[pl.ds(base + (j0 + 1) * CH, CH)], buf1, sem1).wait()
        pltpu.sync_copy(buf1, acc.at[idx_v.at[j0 + 1]], add=True)
        return carry

    lax.fori_loop(0, NCH // 2, step, 0)
    plsc.subcore_barrier()
    pltpu.sync_copy(acc.at[pl.ds(s * CPT, CPT)], out_hbm.at[c, pl.ds(s * CPT, CPT)])


def _gru_big_body(ef_ref, whh_ref, bih_ref, bhh_ref, out_ref):
    h = ef_ref[...]
    gh = jnp.dot(h, whh_ref[...], preferred_element_type=jnp.float32) + bhh_ref[...]
    gi = bih_ref[...]
    r = jax.nn.sigmoid(gi[:, :H] + gh[:, :H])
    z = jax.nn.sigmoid(gi[:, H:2 * H] + gh[:, H:2 * H])
    n = jnp.tanh(gi[:, 2 * H:] + r * gh[:, 2 * H:])
    out_ref[...] = (1.0 - z) * n + z * h


def _gru_small_body(ef_ref, p0_ref, p1_ref, wih_ref, whh_ref, bih_ref,
                    bhh_ref, big_ref, out_ref):
    del big_ref
    h = ef_ref[...]
    gh = jnp.dot(h, whh_ref[...], preferred_element_type=jnp.float32) + bhh_ref[...]
    x = p0_ref[...] + p1_ref[...]
    gi = jnp.dot(x, wih_ref[...], preferred_element_type=jnp.float32) + bih_ref[...]
    r = jax.nn.sigmoid(gi[:, :H] + gh[:, :H])
    z = jax.nn.sigmoid(gi[:, H:2 * H] + gh[:, H:2 * H])
    n = jnp.tanh(gi[:, 2 * H:] + r * gh[:, 2 * H:])
    out_ref[...] = (1.0 - z) * n + z * h


def kernel(edge_features, edge_index, W_msg_w, W_msg_b, W_ih, W_hh, b_ih, b_hh):
    E = edge_features.shape[0]

    last_in_blk = (E - 1) // TA
    message = pl.pallas_call(
        _msg_body,
        grid=(E_PAD // TA,),
        in_specs=[
            pl.BlockSpec((TA, H), lambda i: (jnp.minimum(i, last_in_blk), 0)),
            pl.BlockSpec((H, H), lambda i: (0, 0)),
            pl.BlockSpec((1, H), lambda i: (0, 0)),
        ],
        out_specs=pl.BlockSpec((TA, H), lambda i: (i, 0)),
        out_shape=jax.ShapeDtypeStruct((E_PAD, H), jnp.float32),
    )(edge_features, W_msg_w.T, W_msg_b.reshape(1, H))

    idx = jnp.full((E_PAD,), N_NODES, dtype=jnp.int32).at[:E].set(edge_index[1])
    idx3 = idx.reshape(NW, NCH, CH)
    zeros = jnp.zeros((ZR, H), dtype=jnp.float32)

    mesh = plsc.VectorSubcoreMesh(core_axis_name="c", subcore_axis_name="s")
    scatter = pl.kernel(
        _scatter_body,
        out_type=jax.ShapeDtypeStruct((NC, N_OUT, H), jnp.float32),
        mesh=mesh,
        scratch_types=[
            pltpu.VMEM((NCH, CH), jnp.int32),
            pltpu.VMEM((CH, H), jnp.float32),
            pltpu.VMEM((CH, H), jnp.float32),
            pltpu.VMEM_SHARED((N_ACC, H), jnp.float32),
            pltpu.SemaphoreType.DMA,
            pltpu.SemaphoreType.DMA,
        ],
    )
    partials = scatter(message, idx3, zeros)

    big = pl.pallas_call(
        _gru_big_body,
        grid=((E - N_NODES) // TB,),
        in_specs=[
            pl.BlockSpec((TB, H), lambda i: (i + NT, 0)),
            pl.BlockSpec((H, 3 * H), lambda i: (0, 0)),
            pl.BlockSpec((1, 3 * H), lambda i: (0, 0)),
            pl.BlockSpec((1, 3 * H), lambda i: (0, 0)),
        ],
        out_specs=pl.BlockSpec((TB, H), lambda i: (i + NT, 0)),
        out_shape=jax.ShapeDtypeStruct((E, H), jnp.float32),
    )(edge_features, W_hh.T, b_ih.reshape(1, 3 * H), b_hh.reshape(1, 3 * H))

    out = pl.pallas_call(
        _gru_small_body,
        grid=(NT,),
        in_specs=[
            pl.BlockSpec((TB, H), lambda i: (i, 0)),
            pl.BlockSpec((TB, H), lambda i: (i, 0)),
            pl.BlockSpec((TB, H), lambda i: (i, 0)),
            pl.BlockSpec((H, 3 * H), lambda i: (0, 0)),
            pl.BlockSpec((H, 3 * H), lambda i: (0, 0)),
            pl.BlockSpec((1, 3 * H), lambda i: (0, 0)),
            pl.BlockSpec((1, 3 * H), lambda i: (0, 0)),
            pl.BlockSpec(memory_space=pl.ANY),
        ],
        out_specs=pl.BlockSpec((TB, H), lambda i: (i, 0)),
        out_shape=jax.ShapeDtypeStruct((E, H), jnp.float32),
        input_output_aliases={7: 0},
    )(edge_features, partials[0], partials[1], W_ih.T, W_hh.T,
      b_ih.reshape(1, 3 * H), b_hh.reshape(1, 3 * H), big)
    return out


# speedup vs baseline: 5.1281x; 1.2711x over previous
import functools

import jax
import jax.numpy as jnp
from jax import lax
from jax.experimental import pallas as pl
from jax.experimental.pallas import tpu as pltpu
from jax.experimental.pallas import tpu_sc as plsc

H = 128
N_NODES = 10000
E_TOTAL = 320000

NC = 2
NS = 16
NW = NC * NS

SLAB_B = 96000
SLAB_A = E_TOTAL - SLAB_B

EPW_A = SLAB_A // NW
CH_A = 56
NCH_A = EPW_A // CH_A
EPW_B = SLAB_B // NW
CH_B = 120
NCH_B = EPW_B // CH_B

ZR = 632
N_ACC = ZR * NS

TA = 4000
TB = 2000
NT = N_NODES // TB


def _make_scatter_body(epw, ch, nch):
    def _scatter_body(msg_hbm, idx_hbm, zeros_hbm, out_hbm,
                      idx_v, buf0, buf1, acc, sem0, sem1):
        c = lax.axis_index("c")
        s = lax.axis_index("s")
        wid = c * NS + s
        pltpu.sync_copy(zeros_hbm, acc.at[pl.ds(s * ZR, ZR)])
        pltpu.sync_copy(idx_hbm.at[wid], idx_v)
        plsc.subcore_barrier()

        base = wid * epw
        pltpu.async_copy(msg_hbm.at[pl.ds(base, ch)], buf0, sem0)

        def step(k, carry):
            j0 = 2 * k
            pltpu.async_copy(msg_hbm.at[pl.ds(base + (j0 + 1) * ch, ch)], buf1, sem1)
            pltpu.make_async_copy(msg_hbm.at[pl.ds(base + j0 * ch, ch)], buf0, sem0).wait()
            pltpu.sync_copy(buf0, acc.at[idx_v.at[j0]], add=True)

            @pl.when(j0 + 2 < nch)
            def _():
                pltpu.async_copy(msg_hbm.at[pl.ds(base + (j0 + 2) * ch, ch)], buf0, sem0)

            pltpu.make_async_copy(msg_hbm.at[pl.ds(base + (j0 + 1) * ch, ch)], buf1, sem1).wait()
            pltpu.sync_copy(buf1, acc.at[idx_v.at[j0 + 1]], add=True)
            return carry

        lax.fori_loop(0, nch // 2, step, 0)
        if nch % 2:
            j = nch - 1
            pltpu.make_async_copy(msg_hbm.at[pl.ds(base + j * ch, ch)], buf0, sem0).wait()
            pltpu.sync_copy(buf0, acc.at[idx_v.at[j]], add=True)
        plsc.subcore_barrier()
        pltpu.sync_copy(acc.at[pl.ds(s * ZR, ZR)], out_hbm.at[c, pl.ds(s * ZR, ZR)])

    return _scatter_body


def _sc_scatter(message, idx3, zeros, epw, ch, nch):
    mesh = plsc.VectorSubcoreMesh(core_axis_name="c", subcore_axis_name="s")
    scatter = pl.kernel(
        _make_scatter_body(epw, ch, nch),
        out_type=jax.ShapeDtypeStruct((NC, N_ACC, H), jnp.float32),
        mesh=mesh,
        scratch_types=[
            pltpu.VMEM((nch, ch), jnp.int32),
            pltpu.VMEM((ch, H), jnp.float32),
            pltpu.VMEM((ch, H), jnp.float32),
            pltpu.VMEM_SHARED((N_ACC, H), jnp.float32),
            pltpu.SemaphoreType.DMA,
            pltpu.SemaphoreType.DMA,
        ],
    )
    return scatter(message, idx3, zeros)


def _fused_body(ef_ref, wm_ref, bm_ref, whh_ref, bih_ref, bhh_ref,
                msg_ref, out_ref):
    h = ef_ref[...]
    m = jnp.dot(h, wm_ref[...], preferred_element_type=jnp.float32) + bm_ref[...]
    msg_ref[...] = jnp.maximum(m, 0.0)
    gh = jnp.dot(h, whh_ref[...], preferred_element_type=jnp.float32) + bhh_ref[...]
    gi = bih_ref[...]
    r = jax.nn.sigmoid(gi[:, :H] + gh[:, :H])
    z = jax.nn.sigmoid(gi[:, H:2 * H] + gh[:, H:2 * H])
    n = jnp.tanh(gi[:, 2 * H:] + r * gh[:, 2 * H:])
    out_ref[...] = (1.0 - z) * n + z * h


def _fused_donor_body(ef_ref, wm_ref, bm_ref, whh_ref, bih_ref, bhh_ref,
                      big_ref, msg_ref, out_ref):
    del big_ref
    _fused_body(ef_ref, wm_ref, bm_ref, whh_ref, bih_ref, bhh_ref,
                msg_ref, out_ref)


def _gru_small_body(ef_ref, pa_ref, pb_ref, wih_ref, whh_ref, bih_ref,
                    bhh_ref, big_ref, out_ref):
    del big_ref
    h = ef_ref[...]
    gh = jnp.dot(h, whh_ref[...], preferred_element_type=jnp.float32) + bhh_ref[...]
    x = pa_ref[0] + pa_ref[1] + pb_ref[0] + pb_ref[1]
    gi = jnp.dot(x, wih_ref[...], preferred_element_type=jnp.float32) + bih_ref[...]
    r = jax.nn.sigmoid(gi[:, :H] + gh[:, :H])
    z = jax.nn.sigmoid(gi[:, H:2 * H] + gh[:, H:2 * H])
    n = jnp.tanh(gi[:, 2 * H:] + r * gh[:, 2 * H:])
    out_ref[...] = (1.0 - z) * n + z * h


def kernel(edge_features, edge_index, W_msg_w, W_msg_b, W_ih, W_hh, b_ih, b_hh):
    E = edge_features.shape[0]
    wm = W_msg_w.T
    bm = W_msg_b.reshape(1, H)
    wih = W_ih.T
    whh = W_hh.T
    bih = b_ih.reshape(1, 3 * H)
    bhh = b_hh.reshape(1, 3 * H)

    dest = edge_index[1]
    idxA = dest[SLAB_B:].reshape(NW, NCH_A, CH_A)
    idxB = dest[:SLAB_B].reshape(NW, NCH_B, CH_B)
    zeros = jnp.zeros((ZR, H), dtype=jnp.float32)

    w_specs = [
        pl.BlockSpec((H, H), lambda i: (0, 0)),
        pl.BlockSpec((1, H), lambda i: (0, 0)),
        pl.BlockSpec((H, 3 * H), lambda i: (0, 0)),
        pl.BlockSpec((1, 3 * H), lambda i: (0, 0)),
        pl.BlockSpec((1, 3 * H), lambda i: (0, 0)),
    ]
    offs = SLAB_B // TA

    msgA, out1 = pl.pallas_call(
        _fused_body,
        grid=(SLAB_A // TA,),
        in_specs=[pl.BlockSpec((TA, H), lambda i: (i + offs, 0))] + w_specs,
        out_specs=[
            pl.BlockSpec((TA, H), lambda i: (i, 0)),
            pl.BlockSpec((TA, H), lambda i: (i + offs, 0)),
        ],
        out_shape=[
            jax.ShapeDtypeStruct((SLAB_A, H), jnp.float32),
            jax.ShapeDtypeStruct((E, H), jnp.float32),
        ],
    )(edge_features, wm, bm, whh, bih, bhh)

    partA = _sc_scatter(msgA, idxA, zeros, EPW_A, CH_A, NCH_A)

    msgB, out2 = pl.pallas_call(
        _fused_donor_body,
        grid=(SLAB_B // TA,),
        in_specs=[pl.BlockSpec((TA, H), lambda i: (i, 0))] + w_specs
        + [pl.BlockSpec(memory_space=pl.ANY)],
        out_specs=[
            pl.BlockSpec((TA, H), lambda i: (i, 0)),
            pl.BlockSpec((TA, H), lambda i: (i, 0)),
        ],
        out_shape=[
            jax.ShapeDtypeStruct((SLAB_B, H), jnp.float32),
            jax.ShapeDtypeStruct((E, H), jnp.float32),
        ],
        input_output_aliases={6: 1},
    )(edge_features, wm, bm, whh, bih, bhh, out1)

    partB = _sc_scatter(msgB, idxB, zeros, EPW_B, CH_B, NCH_B)

    out = pl.pallas_call(
        _gru_small_body,
        grid=(NT,),
        in_specs=[
            pl.BlockSpec((TB, H), lambda i: (i, 0)),
            pl.BlockSpec((NC, TB, H), lambda i: (0, i, 0)),
            pl.BlockSpec((NC, TB, H), lambda i: (0, i, 0)),
            pl.BlockSpec((H, 3 * H), lambda i: (0, 0)),
            pl.BlockSpec((H, 3 * H), lambda i: (0, 0)),
            pl.BlockSpec((1, 3 * H), lambda i: (0, 0)),
            pl.BlockSpec((1, 3 * H), lambda i: (0, 0)),
            pl.BlockSpec(memory_space=pl.ANY),
        ],
        out_specs=pl.BlockSpec((TB, H), lambda i: (i, 0)),
        out_shape=jax.ShapeDtypeStruct((E, H), jnp.float32),
        input_output_aliases={7: 0},
    )(edge_features, partA, partB, wih, whh, bih, bhh, out2)
    return out


# speedup vs baseline: 5.1907x; 1.0122x over previous
import functools

import jax
import jax.numpy as jnp
from jax import lax
from jax.experimental import pallas as pl
from jax.experimental.pallas import tpu as pltpu
from jax.experimental.pallas import tpu_sc as plsc

H = 128
N_NODES = 10000
E_TOTAL = 320000

NC = 2
NS = 16
NW = NC * NS

SLAB_B = 102400
SLAB_A = E_TOTAL - SLAB_B

CH = 128
EPW_A = SLAB_A // NW
NCH_A = -(-EPW_A // CH)
EPW_B = SLAB_B // NW
NCH_B = -(-EPW_B // CH) + 1

ZR = 632
N_ACC = ZR * NS

TA = 3200
TB = 2000
NT = N_NODES // TB


def _make_scatter_body(epw, nch):
    def _scatter_body(msg_hbm, idx_hbm, zeros_hbm, out_hbm,
                      idx_v, buf0, buf1, acc, sem0, sem1):
        c = lax.axis_index("c")
        s = lax.axis_index("s")
        wid = c * NS + s
        pltpu.sync_copy(zeros_hbm, acc.at[pl.ds(s * ZR, ZR)])
        pltpu.sync_copy(idx_hbm.at[wid], idx_v)
        plsc.subcore_barrier()

        base = wid * epw
        pltpu.async_copy(msg_hbm.at[pl.ds(base, CH)], buf0, sem0)

        def step(k, carry):
            j0 = 2 * k
            pltpu.async_copy(msg_hbm.at[pl.ds(base + (j0 + 1) * CH, CH)], buf1, sem1)
            pltpu.make_async_copy(msg_hbm.at[pl.ds(base + j0 * CH, CH)], buf0, sem0).wait()
            pltpu.sync_copy(buf0, acc.at[idx_v.at[j0]], add=True)

            @pl.when(j0 + 2 < nch)
            def _():
                pltpu.async_copy(msg_hbm.at[pl.ds(base + (j0 + 2) * CH, CH)], buf0, sem0)

            pltpu.make_async_copy(msg_hbm.at[pl.ds(base + (j0 + 1) * CH, CH)], buf1, sem1).wait()
            pltpu.sync_copy(buf1, acc.at[idx_v.at[j0 + 1]], add=True)
            return carry

        lax.fori_loop(0, nch // 2, step, 0)
        plsc.subcore_barrier()
        pltpu.sync_copy(acc.at[pl.ds(s * ZR, ZR)], out_hbm.at[c, pl.ds(s * ZR, ZR)])

    return _scatter_body


def _sc_scatter(message, idx3, zeros, epw, nch):
    mesh = plsc.VectorSubcoreMesh(core_axis_name="c", subcore_axis_name="s")
    scatter = pl.kernel(
        _make_scatter_body(epw, nch),
        out_type=jax.ShapeDtypeStruct((NC, N_ACC, H), jnp.float32),
        mesh=mesh,
        scratch_types=[
            pltpu.VMEM((nch, CH), jnp.int32),
            pltpu.VMEM((CH, H), jnp.float32),
            pltpu.VMEM((CH, H), jnp.float32),
            pltpu.VMEM_SHARED((N_ACC, H), jnp.float32),
            pltpu.SemaphoreType.DMA,
            pltpu.SemaphoreType.DMA,
        ],
    )
    return scatter(message, idx3, zeros)


def _pad_idx(dest_slice, epw, nch):
    d2 = dest_slice.reshape(NW, epw)
    d2 = jnp.pad(d2, ((0, 0), (0, nch * CH - epw)), constant_values=N_NODES)
    return d2.reshape(NW, nch, CH)


def _fused_body(ef_ref, wm_ref, bm_ref, whh_ref, bih_ref, bhh_ref,
                msg_ref, out_ref):
    h = ef_ref[...]
    m = jnp.dot(h, wm_ref[...], preferred_element_type=jnp.float32) + bm_ref[...]
    msg_ref[...] = jnp.maximum(m, 0.0)
    gh = jnp.dot(h, whh_ref[...], preferred_element_type=jnp.float32) + bhh_ref[...]
    gi = bih_ref[...]
    r = jax.nn.sigmoid(gi[:, :H] + gh[:, :H])
    z = jax.nn.sigmoid(gi[:, H:2 * H] + gh[:, H:2 * H])
    n = jnp.tanh(gi[:, 2 * H:] + r * gh[:, 2 * H:])
    out_ref[...] = (1.0 - z) * n + z * h


def _fused_donor_body(ef_ref, wm_ref, bm_ref, whh_ref, bih_ref, bhh_ref,
                      big_ref, msg_ref, out_ref):
    del big_ref
    _fused_body(ef_ref, wm_ref, bm_ref, whh_ref, bih_ref, bhh_ref,
                msg_ref, out_ref)


def _gru_small_body(ef_ref, pa_ref, pb_ref, wih_ref, whh_ref, bih_ref,
                    bhh_ref, big_ref, out_ref):
    del big_ref
    h = ef_ref[...]
    gh = jnp.dot(h, whh_ref[...], preferred_element_type=jnp.float32) + bhh_ref[...]
    x = pa_ref[0] + pa_ref[1] + pb_ref[0] + pb_ref[1]
    gi = jnp.dot(x, wih_ref[...], preferred_element_type=jnp.float32) + bih_ref[...]
    r = jax.nn.sigmoid(gi[:, :H] + gh[:, :H])
    z = jax.nn.sigmoid(gi[:, H:2 * H] + gh[:, H:2 * H])
    n = jnp.tanh(gi[:, 2 * H:] + r * gh[:, 2 * H:])
    out_ref[...] = (1.0 - z) * n + z * h


def kernel(edge_features, edge_index, W_msg_w, W_msg_b, W_ih, W_hh, b_ih, b_hh):
    E = edge_features.shape[0]
    wm = W_msg_w.T
    bm = W_msg_b.reshape(1, H)
    wih = W_ih.T
    whh = W_hh.T
    bih = b_ih.reshape(1, 3 * H)
    bhh = b_hh.reshape(1, 3 * H)

    dest = edge_index[1]
    idxA = _pad_idx(dest[SLAB_B:], EPW_A, NCH_A)
    idxB = _pad_idx(dest[:SLAB_B], EPW_B, NCH_B)
    zeros = jnp.zeros((ZR, H), dtype=jnp.float32)

    w_specs = [
        pl.BlockSpec((H, H), lambda i: (0, 0)),
        pl.BlockSpec((1, H), lambda i: (0, 0)),
        pl.BlockSpec((H, 3 * H), lambda i: (0, 0)),
        pl.BlockSpec((1, 3 * H), lambda i: (0, 0)),
        pl.BlockSpec((1, 3 * H), lambda i: (0, 0)),
    ]
    offs = SLAB_B // TA

    msgA, out1 = pl.pallas_call(
        _fused_body,
        grid=(SLAB_A // TA,),
        in_specs=[pl.BlockSpec((TA, H), lambda i: (i + offs, 0))] + w_specs,
        out_specs=[
            pl.BlockSpec((TA, H), lambda i: (i, 0)),
            pl.BlockSpec((TA, H), lambda i: (i + offs, 0)),
        ],
        out_shape=[
            jax.ShapeDtypeStruct((SLAB_A + CH, H), jnp.float32),
            jax.ShapeDtypeStruct((E, H), jnp.float32),
        ],
    )(edge_features, wm, bm, whh, bih, bhh)

    partA = _sc_scatter(msgA, idxA, zeros, EPW_A, NCH_A)

    msgB, out2 = pl.pallas_call(
        _fused_donor_body,
        grid=(SLAB_B // TA,),
        in_specs=[pl.BlockSpec((TA, H), lambda i: (i, 0))] + w_specs
        + [pl.BlockSpec(memory_space=pl.ANY)],
        out_specs=[
            pl.BlockSpec((TA, H), lambda i: (i, 0)),
            pl.BlockSpec((TA, H), lambda i: (i, 0)),
        ],
        out_shape=[
            jax.ShapeDtypeStruct((SLAB_B + CH, H), jnp.float32),
            jax.ShapeDtypeStruct((E, H), jnp.float32),
        ],
        input_output_aliases={6: 1},
    )(edge_features, wm, bm, whh, bih, bhh, out1)

    partB = _sc_scatter(msgB, idxB, zeros, EPW_B, NCH_B)

    out = pl.pallas_call(
        _gru_small_body,
        grid=(NT,),
        in_specs=[
            pl.BlockSpec((TB, H), lambda i: (i, 0)),
            pl.BlockSpec((NC, TB, H), lambda i: (0, i, 0)),
            pl.BlockSpec((NC, TB, H), lambda i: (0, i, 0)),
            pl.BlockSpec((H, 3 * H), lambda i: (0, 0)),
            pl.BlockSpec((H, 3 * H), lambda i: (0, 0)),
            pl.BlockSpec((1, 3 * H), lambda i: (0, 0)),
            pl.BlockSpec((1, 3 * H), lambda i: (0, 0)),
            pl.BlockSpec(memory_space=pl.ANY),
        ],
        out_specs=pl.BlockSpec((TB, H), lambda i: (i, 0)),
        out_shape=jax.ShapeDtypeStruct((E, H), jnp.float32),
        input_output_aliases={7: 0},
    )(edge_features, partA, partB, wih, whh, bih, bhh, out2)
    return out


# speedup vs baseline: 5.3362x; 1.0280x over previous
import functools

import jax
import jax.numpy as jnp
from jax import lax
from jax.experimental import pallas as pl
from jax.experimental.pallas import tpu as pltpu
from jax.experimental.pallas import tpu_sc as plsc

H = 128
N_NODES = 10000
E_TOTAL = 320000

NC = 2
NS = 16
NW = NC * NS

SLABS = (140800, 102400, 76800)
CH = 128

ZR = 632
N_ACC = ZR * NS
ZFULL = ZR // CH
ZTAIL = ZR - ZFULL * CH

TA = 3200
TB = 2000
NT = N_NODES // TB


def _make_scatter_body(epw, nch):
    def _scatter_body(msg_hbm, idx_hbm, zeros_hbm, out_hbm,
                      idx_v, buf0, buf1, acc, sem0, sem1, semi):
        c = lax.axis_index("c")
        s = lax.axis_index("s")
        wid = c * NS + s
        base = wid * epw

        pltpu.async_copy(idx_hbm.at[wid], idx_v, semi)
        pltpu.sync_copy(zeros_hbm, buf0)
        for t in range(ZFULL):
            pltpu.sync_copy(buf0, acc.at[pl.ds(s * ZR + t * CH, CH)])
        pltpu.sync_copy(buf0.at[pl.ds(0, ZTAIL)],
                        acc.at[pl.ds(s * ZR + ZFULL * CH, ZTAIL)])
        pltpu.async_copy(msg_hbm.at[pl.ds(base, CH)], buf0, sem0)
        pltpu.make_async_copy(idx_hbm.at[wid], idx_v, semi).wait()
        plsc.subcore_barrier()

        def step(k, carry):
            j0 = 2 * k
            pltpu.async_copy(msg_hbm.at[pl.ds(base + (j0 + 1) * CH, CH)], buf1, sem1)
            pltpu.make_async_copy(msg_hbm.at[pl.ds(base + j0 * CH, CH)], buf0, sem0).wait()
            pltpu.sync_copy(buf0, acc.at[idx_v.at[j0]], add=True)

            @pl.when(j0 + 2 < nch)
            def _():
                pltpu.async_copy(msg_hbm.at[pl.ds(base + (j0 + 2) * CH, CH)], buf0, sem0)

            pltpu.make_async_copy(msg_hbm.at[pl.ds(base + (j0 + 1) * CH, CH)], buf1, sem1).wait()
            pltpu.sync_copy(buf1, acc.at[idx_v.at[j0 + 1]], add=True)
            return carry

        lax.fori_loop(0, nch // 2, step, 0)
        plsc.subcore_barrier()
        pltpu.sync_copy(acc.at[pl.ds(s * ZR, ZR)], out_hbm.at[c, pl.ds(s * ZR, ZR)])

    return _scatter_body


def _sc_scatter(message, idx3, zeros, epw, nch):
    mesh = plsc.VectorSubcoreMesh(core_axis_name="c", subcore_axis_name="s")
    scatter = pl.kernel(
        _make_scatter_body(epw, nch),
        out_type=jax.ShapeDtypeStruct((NC, N_ACC, H), jnp.float32),
        mesh=mesh,
        scratch_types=[
            pltpu.VMEM((nch, CH), jnp.int32),
            pltpu.VMEM((CH, H), jnp.float32),
            pltpu.VMEM((CH, H), jnp.float32),
            pltpu.VMEM_SHARED((N_ACC, H), jnp.float32),
            pltpu.SemaphoreType.DMA,
            pltpu.SemaphoreType.DMA,
            pltpu.SemaphoreType.DMA,
        ],
    )
    return scatter(message, idx3, zeros)


def _pad_idx(dest_slice, epw, nch):
    d2 = dest_slice.reshape(NW, epw)
    d2 = jnp.pad(d2, ((0, 0), (0, nch * CH - epw)), constant_values=N_NODES)
    return d2.reshape(NW, nch, CH)


def _fused_body(ef_ref, wm_ref, bm_ref, whh_ref, bih_ref, bhh_ref,
                msg_ref, out_ref):
    h = ef_ref[...]
    m = jnp.dot(h, wm_ref[...], preferred_element_type=jnp.float32) + bm_ref[...]
    msg_ref[...] = jnp.maximum(m, 0.0)
    gh = jnp.dot(h, whh_ref[...], preferred_element_type=jnp.float32) + bhh_ref[...]
    gi = bih_ref[...]
    r = jax.nn.sigmoid(gi[:, :H] + gh[:, :H])
    z = jax.nn.sigmoid(gi[:, H:2 * H] + gh[:, H:2 * H])
    n = jnp.tanh(gi[:, 2 * H:] + r * gh[:, 2 * H:])
    out_ref[...] = (1.0 - z) * n + z * h


def _fused_donor_body(ef_ref, wm_ref, bm_ref, whh_ref, bih_ref, bhh_ref,
                      big_ref, msg_ref, out_ref):
    del big_ref
    _fused_body(ef_ref, wm_ref, bm_ref, whh_ref, bih_ref, bhh_ref,
                msg_ref, out_ref)


def _gru_small_body(ef_ref, p0_ref, p1_ref, p2_ref, wih_ref, whh_ref,
                    bih_ref, bhh_ref, big_ref, out_ref):
    del big_ref
    h = ef_ref[...]
    gh = jnp.dot(h, whh_ref[...], preferred_element_type=jnp.float32) + bhh_ref[...]
    x = (p0_ref[0] + p0_ref[1]) + (p1_ref[0] + p1_ref[1]) + (p2_ref[0] + p2_ref[1])
    gi = jnp.dot(x, wih_ref[...], preferred_element_type=jnp.float32) + bih_ref[...]
    r = jax.nn.sigmoid(gi[:, :H] + gh[:, :H])
    z = jax.nn.sigmoid(gi[:, H:2 * H] + gh[:, H:2 * H])
    n = jnp.tanh(gi[:, 2 * H:] + r * gh[:, 2 * H:])
    out_ref[...] = (1.0 - z) * n + z * h


def kernel(edge_features, edge_index, W_msg_w, W_msg_b, W_ih, W_hh, b_ih, b_hh):
    E = edge_features.shape[0]
    wm = W_msg_w.T
    bm = W_msg_b.reshape(1, H)
    wih = W_ih.T
    whh = W_hh.T
    bih = b_ih.reshape(1, 3 * H)
    bhh = b_hh.reshape(1, 3 * H)

    dest = edge_index[1]
    zeros = jnp.zeros((CH, H), dtype=jnp.float32)

    w_specs = [
        pl.BlockSpec((H, H), lambda i: (0, 0)),
        pl.BlockSpec((1, H), lambda i: (0, 0)),
        pl.BlockSpec((H, 3 * H), lambda i: (0, 0)),
        pl.BlockSpec((1, 3 * H), lambda i: (0, 0)),
        pl.BlockSpec((1, 3 * H), lambda i: (0, 0)),
    ]

    out = None
    partials = []
    start = 0
    for slab in SLABS:
        epw = slab // NW
        nch = -(-epw // CH)
        if nch % 2:
            nch += 1
        offs = start // TA

        in_specs = [pl.BlockSpec((TA, H),
                                 functools.partial(lambda o, i: (i + o, 0), offs))]
        in_specs += w_specs
        args = [edge_features, wm, bm, whh, bih, bhh]
        body = _fused_body
        aliases = {}
        if out is not None:
            in_specs += [pl.BlockSpec(memory_space=pl.ANY)]
            args += [out]
            body = _fused_donor_body
            aliases = {6: 1}

        msg, out = pl.pallas_call(
            body,
            grid=(slab // TA,),
            in_specs=in_specs,
            out_specs=[
                pl.BlockSpec((TA, H), lambda i: (i, 0)),
                pl.BlockSpec((TA, H),
                             functools.partial(lambda o, i: (i + o, 0), offs)),
            ],
            out_shape=[
                jax.ShapeDtypeStruct((slab + nch * CH - epw, H), jnp.float32),
                jax.ShapeDtypeStruct((E, H), jnp.float32),
            ],
            input_output_aliases=aliases,
        )(*args)

        idx3 = _pad_idx(dest[start:start + slab], epw, nch)
        partials.append(_sc_scatter(msg, idx3, zeros, epw, nch))
        start += slab

    out = pl.pallas_call(
        _gru_small_body,
        grid=(NT,),
        in_specs=[
            pl.BlockSpec((TB, H), lambda i: (i, 0)),
            pl.BlockSpec((NC, TB, H), lambda i: (0, i, 0)),
            pl.BlockSpec((NC, TB, H), lambda i: (0, i, 0)),
            pl.BlockSpec((NC, TB, H), lambda i: (0, i, 0)),
            pl.BlockSpec((H, 3 * H), lambda i: (0, 0)),
            pl.BlockSpec((H, 3 * H), lambda i: (0, 0)),
            pl.BlockSpec((1, 3 * H), lambda i: (0, 0)),
            pl.BlockSpec((1, 3 * H), lambda i: (0, 0)),
            pl.BlockSpec(memory_space=pl.ANY),
        ],
        out_specs=pl.BlockSpec((TB, H), lambda i: (i, 0)),
        out_shape=jax.ShapeDtypeStruct((E, H), jnp.float32),
        input_output_aliases={8: 0},
    )(edge_features, partials[0], partials[1], partials[2],
      wih, whh, bih, bhh, out)
    return out


# speedup vs baseline: 5.4987x; 1.0305x over previous
import functools

import jax
import jax.numpy as jnp
from jax import lax
from jax.experimental import pallas as pl
from jax.experimental.pallas import tpu as pltpu
from jax.experimental.pallas import tpu_sc as plsc

H = 128
N_NODES = 10000
E_TOTAL = 320000

NC = 2
NS = 16
NW = NC * NS

SLABS = (140800, 102400, 76800)
CH = 128

ZR = 632
N_ACC = ZR * NS
ZFULL = ZR // CH
ZTAIL = ZR - ZFULL * CH

TA = 3200
TB = 2000
NT = N_NODES // TB


def _make_scatter_body(epw, nch, seeded):
    def _scatter_body(msg_hbm, idx_hbm, init_hbm, out_hbm,
                      idx_v, buf0, buf1, acc, sem0, sem1, semi):
        c = lax.axis_index("c")
        s = lax.axis_index("s")
        wid = c * NS + s
        base = wid * epw

        pltpu.async_copy(idx_hbm.at[wid], idx_v, semi)
        if seeded:
            pltpu.sync_copy(init_hbm.at[c, pl.ds(s * ZR, ZR)],
                            acc.at[pl.ds(s * ZR, ZR)])
        else:
            pltpu.sync_copy(init_hbm, buf0)
            for t in range(ZFULL):
                pltpu.sync_copy(buf0, acc.at[pl.ds(s * ZR + t * CH, CH)])
            pltpu.sync_copy(buf0.at[pl.ds(0, ZTAIL)],
                            acc.at[pl.ds(s * ZR + ZFULL * CH, ZTAIL)])
        pltpu.async_copy(msg_hbm.at[pl.ds(base, CH)], buf0, sem0)
        pltpu.make_async_copy(idx_hbm.at[wid], idx_v, semi).wait()
        plsc.subcore_barrier()

        def step(k, carry):
            j0 = 2 * k
            pltpu.async_copy(msg_hbm.at[pl.ds(base + (j0 + 1) * CH, CH)], buf1, sem1)
            pltpu.make_async_copy(msg_hbm.at[pl.ds(base + j0 * CH, CH)], buf0, sem0).wait()
            pltpu.sync_copy(buf0, acc.at[idx_v.at[j0]], add=True)

            @pl.when(j0 + 2 < nch)
            def _():
                pltpu.async_copy(msg_hbm.at[pl.ds(base + (j0 + 2) * CH, CH)], buf0, sem0)

            pltpu.make_async_copy(msg_hbm.at[pl.ds(base + (j0 + 1) * CH, CH)], buf1, sem1).wait()
            pltpu.sync_copy(buf1, acc.at[idx_v.at[j0 + 1]], add=True)
            return carry

        lax.fori_loop(0, nch // 2, step, 0)
        plsc.subcore_barrier()
        pltpu.sync_copy(acc.at[pl.ds(s * ZR, ZR)], out_hbm.at[c, pl.ds(s * ZR, ZR)])

    return _scatter_body


def _sc_scatter(message, idx3, init, epw, nch, seeded):
    mesh = plsc.VectorSubcoreMesh(core_axis_name="c", subcore_axis_name="s")
    scatter = pl.kernel(
        _make_scatter_body(epw, nch, seeded),
        out_type=jax.ShapeDtypeStruct((NC, N_ACC, H), jnp.float32),
        mesh=mesh,
        scratch_types=[
            pltpu.VMEM((nch, CH), jnp.int32),
            pltpu.VMEM((CH, H), jnp.float32),
            pltpu.VMEM((CH, H), jnp.float32),
            pltpu.VMEM_SHARED((N_ACC, H), jnp.float32),
            pltpu.SemaphoreType.DMA,
            pltpu.SemaphoreType.DMA,
            pltpu.SemaphoreType.DMA,
        ],
    )
    return scatter(message, idx3, init)


def _pad_idx(dest_slice, epw, nch):
    d2 = dest_slice.reshape(NW, epw)
    d2 = jnp.pad(d2, ((0, 0), (0, nch * CH - epw)), constant_values=N_NODES)
    return d2.reshape(NW, nch, CH)


def _fused_body(ef_ref, wm_ref, bm_ref, whh_ref, bih_ref, bhh_ref,
                msg_ref, out_ref):
    h = ef_ref[...]
    m = jnp.dot(h, wm_ref[...], preferred_element_type=jnp.float32) + bm_ref[...]
    msg_ref[...] = jnp.maximum(m, 0.0)
    gh = jnp.dot(h, whh_ref[...], preferred_element_type=jnp.float32) + bhh_ref[...]
    gi = bih_ref[...]
    r = jax.nn.sigmoid(gi[:, :H] + gh[:, :H])
    z = jax.nn.sigmoid(gi[:, H:2 * H] + gh[:, H:2 * H])
    n = jnp.tanh(gi[:, 2 * H:] + r * gh[:, 2 * H:])
    out_ref[...] = (1.0 - z) * n + z * h


def _fused_donor_body(ef_ref, wm_ref, bm_ref, whh_ref, bih_ref, bhh_ref,
                      big_ref, msg_ref, out_ref):
    del big_ref
    _fused_body(ef_ref, wm_ref, bm_ref, whh_ref, bih_ref, bhh_ref,
                msg_ref, out_ref)


def _gru_small_body(ef_ref, p_ref, wih_ref, whh_ref,
                    bih_ref, bhh_ref, big_ref, out_ref):
    del big_ref
    h = ef_ref[...]
    gh = jnp.dot(h, whh_ref[...], preferred_element_type=jnp.float32) + bhh_ref[...]
    x = p_ref[0] + p_ref[1]
    gi = jnp.dot(x, wih_ref[...], preferred_element_type=jnp.float32) + bih_ref[...]
    r = jax.nn.sigmoid(gi[:, :H] + gh[:, :H])
    z = jax.nn.sigmoid(gi[:, H:2 * H] + gh[:, H:2 * H])
    n = jnp.tanh(gi[:, 2 * H:] + r * gh[:, 2 * H:])
    out_ref[...] = (1.0 - z) * n + z * h


def kernel(edge_features, edge_index, W_msg_w, W_msg_b, W_ih, W_hh, b_ih, b_hh):
    E = edge_features.shape[0]
    wm = W_msg_w.T
    bm = W_msg_b.reshape(1, H)
    wih = W_ih.T
    whh = W_hh.T
    bih = b_ih.reshape(1, 3 * H)
    bhh = b_hh.reshape(1, 3 * H)

    dest = edge_index[1]
    zeros = jnp.zeros((CH, H), dtype=jnp.float32)

    w_specs = [
        pl.BlockSpec((H, H), lambda i: (0, 0)),
        pl.BlockSpec((1, H), lambda i: (0, 0)),
        pl.BlockSpec((H, 3 * H), lambda i: (0, 0)),
        pl.BlockSpec((1, 3 * H), lambda i: (0, 0)),
        pl.BlockSpec((1, 3 * H), lambda i: (0, 0)),
    ]

    out = None
    part = zeros
    seeded = False
    start = 0
    for slab in SLABS:
        epw = slab // NW
        nch = -(-epw // CH)
        if nch % 2:
            nch += 1
        offs = start // TA

        in_specs = [pl.BlockSpec((TA, H),
                                 functools.partial(lambda o, i: (i + o, 0), offs))]
        in_specs += w_specs
        args = [edge_features, wm, bm, whh, bih, bhh]
        body = _fused_body
        aliases = {}
        if out is not None:
            in_specs += [pl.BlockSpec(memory_space=pl.ANY)]
            args += [out]
            body = _fused_donor_body
            aliases = {6: 1}

        msg, out = pl.pallas_call(
            body,
            grid=(slab // TA,),
            in_specs=in_specs,
            out_specs=[
                pl.BlockSpec((TA, H), lambda i: (i, 0)),
                pl.BlockSpec((TA, H),
                             functools.partial(lambda o, i: (i + o, 0), offs)),
            ],
            out_shape=[
                jax.ShapeDtypeStruct((slab + nch * CH - epw, H), jnp.float32),
                jax.ShapeDtypeStruct((E, H), jnp.float32),
            ],
            input_output_aliases=aliases,
        )(*args)

        idx3 = _pad_idx(dest[start:start + slab], epw, nch)
        part = _sc_scatter(msg, idx3, part, epw, nch, seeded)
        seeded = True
        start += slab

    out = pl.pallas_call(
        _gru_small_body,
        grid=(NT,),
        in_specs=[
            pl.BlockSpec((TB, H), lambda i: (i, 0)),
            pl.BlockSpec((NC, TB, H), lambda i: (0, i, 0)),
            pl.BlockSpec((H, 3 * H), lambda i: (0, 0)),
            pl.BlockSpec((H, 3 * H), lambda i: (0, 0)),
            pl.BlockSpec((1, 3 * H), lambda i: (0, 0)),
            pl.BlockSpec((1, 3 * H), lambda i: (0, 0)),
            pl.BlockSpec(memory_space=pl.ANY),
        ],
        out_specs=pl.BlockSpec((TB, H), lambda i: (i, 0)),
        out_shape=jax.ShapeDtypeStruct((E, H), jnp.float32),
        input_output_aliases={6: 0},
    )(edge_features, part, wih, whh, bih, bhh, out)
    return out


# speedup vs baseline: 5.6972x; 1.0361x over previous
import functools

import jax
import jax.numpy as jnp
from jax import lax
from jax.experimental import pallas as pl
from jax.experimental.pallas import tpu as pltpu
from jax.experimental.pallas import tpu_sc as plsc

H = 128
N_NODES = 10000
E_TOTAL = 320000

NC = 2
NS = 16
NW = NC * NS

SLABS = (140800, 102400, 76800)
CH = 128

ZR = 632
N_ACC = ZR * NS
ZFULL = ZR // CH
ZTAIL = ZR - ZFULL * CH

TA = 6400
TB = 2000
NT = N_NODES // TB


def _make_scatter_body(epw, nch, seeded):
    def _scatter_body(msg_hbm, idx_hbm, init_hbm, out_hbm,
                      idx_v, buf0, buf1, acc, sem0, sem1, semi):
        c = lax.axis_index("c")
        s = lax.axis_index("s")
        wid = c * NS + s
        base = wid * epw

        pltpu.async_copy(idx_hbm.at[wid], idx_v, semi)
        if seeded:
            pltpu.sync_copy(init_hbm.at[c, pl.ds(s * ZR, ZR)],
                            acc.at[pl.ds(s * ZR, ZR)])
        else:
            pltpu.sync_copy(init_hbm, buf0)
            for t in range(ZFULL):
                pltpu.sync_copy(buf0, acc.at[pl.ds(s * ZR + t * CH, CH)])
            pltpu.sync_copy(buf0.at[pl.ds(0, ZTAIL)],
                            acc.at[pl.ds(s * ZR + ZFULL * CH, ZTAIL)])
        pltpu.async_copy(msg_hbm.at[pl.ds(base, CH)], buf0, sem0)
        pltpu.make_async_copy(idx_hbm.at[wid], idx_v, semi).wait()
        plsc.subcore_barrier()

        def step(k, carry):
            j0 = 2 * k
            pltpu.async_copy(msg_hbm.at[pl.ds(base + (j0 + 1) * CH, CH)], buf1, sem1)
            pltpu.make_async_copy(msg_hbm.at[pl.ds(base + j0 * CH, CH)], buf0, sem0).wait()
            pltpu.sync_copy(buf0, acc.at[idx_v.at[j0]], add=True)

            @pl.when(j0 + 2 < nch)
            def _():
                pltpu.async_copy(msg_hbm.at[pl.ds(base + (j0 + 2) * CH, CH)], buf0, sem0)

            pltpu.make_async_copy(msg_hbm.at[pl.ds(base + (j0 + 1) * CH, CH)], buf1, sem1).wait()
            pltpu.sync_copy(buf1, acc.at[idx_v.at[j0 + 1]], add=True)
            return carry

        lax.fori_loop(0, nch // 2, step, 0)
        plsc.subcore_barrier()
        pltpu.sync_copy(acc.at[pl.ds(s * ZR, ZR)], out_hbm.at[c, pl.ds(s * ZR, ZR)])

    return _scatter_body


def _sc_scatter(message, idx3, init, epw, nch, seeded):
    mesh = plsc.VectorSubcoreMesh(core_axis_name="c", subcore_axis_name="s")
    scatter = pl.kernel(
        _make_scatter_body(epw, nch, seeded),
        out_type=jax.ShapeDtypeStruct((NC, N_ACC, H), jnp.float32),
        mesh=mesh,
        scratch_types=[
            pltpu.VMEM((nch, CH), jnp.int32),
            pltpu.VMEM((CH, H), jnp.float32),
            pltpu.VMEM((CH, H), jnp.float32),
            pltpu.VMEM_SHARED((N_ACC, H), jnp.float32),
            pltpu.SemaphoreType.DMA,
            pltpu.SemaphoreType.DMA,
            pltpu.SemaphoreType.DMA,
        ],
    )
    return scatter(message, idx3, init)


def _pad_idx(dest_slice, epw, nch):
    d2 = dest_slice.reshape(NW, epw)
    d2 = jnp.pad(d2, ((0, 0), (0, nch * CH - epw)), constant_values=N_NODES)
    return d2.reshape(NW, nch, CH)


def _fused_body(ef_ref, wm_ref, bm_ref, whh_ref, bih_ref, bhh_ref,
                msg_ref, out_ref):
    h = ef_ref[...]
    m = jnp.dot(h, wm_ref[...], preferred_element_type=jnp.float32) + bm_ref[...]
    msg_ref[...] = jnp.maximum(m, 0.0)
    gh = jnp.dot(h, whh_ref[...], preferred_element_type=jnp.float32) + bhh_ref[...]
    gi = bih_ref[...]
    r = jax.nn.sigmoid(gi[:, :H] + gh[:, :H])
    z = jax.nn.sigmoid(gi[:, H:2 * H] + gh[:, H:2 * H])
    n = jnp.tanh(gi[:, 2 * H:] + r * gh[:, 2 * H:])
    out_ref[...] = (1.0 - z) * n + z * h


def _fused_donor_body(ef_ref, wm_ref, bm_ref, whh_ref, bih_ref, bhh_ref,
                      big_ref, msg_ref, out_ref):
    del big_ref
    _fused_body(ef_ref, wm_ref, bm_ref, whh_ref, bih_ref, bhh_ref,
                msg_ref, out_ref)


def _gru_small_body(ef_ref, p_ref, wih_ref, whh_ref,
                    bih_ref, bhh_ref, big_ref, out_ref):
    del big_ref
    h = ef_ref[...]
    gh = jnp.dot(h, whh_ref[...], preferred_element_type=jnp.float32) + bhh_ref[...]
    x = p_ref[0] + p_ref[1]
    gi = jnp.dot(x, wih_ref[...], preferred_element_type=jnp.float32) + bih_ref[...]
    r = jax.nn.sigmoid(gi[:, :H] + gh[:, :H])
    z = jax.nn.sigmoid(gi[:, H:2 * H] + gh[:, H:2 * H])
    n = jnp.tanh(gi[:, 2 * H:] + r * gh[:, 2 * H:])
    out_ref[...] = (1.0 - z) * n + z * h


def kernel(edge_features, edge_index, W_msg_w, W_msg_b, W_ih, W_hh, b_ih, b_hh):
    E = edge_features.shape[0]
    wm = W_msg_w.T
    bm = W_msg_b.reshape(1, H)
    wih = W_ih.T
    whh = W_hh.T
    bih = b_ih.reshape(1, 3 * H)
    bhh = b_hh.reshape(1, 3 * H)

    dest = edge_index[1]
    zeros = jnp.zeros((CH, H), dtype=jnp.float32)

    w_specs = [
        pl.BlockSpec((H, H), lambda i: (0, 0)),
        pl.BlockSpec((1, H), lambda i: (0, 0)),
        pl.BlockSpec((H, 3 * H), lambda i: (0, 0)),
        pl.BlockSpec((1, 3 * H), lambda i: (0, 0)),
        pl.BlockSpec((1, 3 * H), lambda i: (0, 0)),
    ]

    out = None
    part = zeros
    seeded = False
    start = 0
    for slab in SLABS:
        epw = slab // NW
        nch = -(-epw // CH)
        if nch % 2:
            nch += 1
        offs = start // TA

        in_specs = [pl.BlockSpec((TA, H),
                                 functools.partial(lambda o, i: (i + o, 0), offs))]
        in_specs += w_specs
        args = [edge_features, wm, bm, whh, bih, bhh]
        body = _fused_body
        aliases = {}
        if out is not None:
            in_specs += [pl.BlockSpec(memory_space=pl.ANY)]
            args += [out]
            body = _fused_donor_body
            aliases = {6: 1}

        msg, out = pl.pallas_call(
            body,
            grid=(slab // TA,),
            in_specs=in_specs,
            out_specs=[
                pl.BlockSpec((TA, H), lambda i: (i, 0)),
                pl.BlockSpec((TA, H),
                             functools.partial(lambda o, i: (i + o, 0), offs)),
            ],
            out_shape=[
                jax.ShapeDtypeStruct((slab + nch * CH - epw, H), jnp.float32),
                jax.ShapeDtypeStruct((E, H), jnp.float32),
            ],
            input_output_aliases=aliases,
        )(*args)

        idx3 = _pad_idx(dest[start:start + slab], epw, nch)
        part = _sc_scatter(msg, idx3, part, epw, nch, seeded)
        seeded = True
        start += slab

    out = pl.pallas_call(
        _gru_small_body,
        grid=(NT,),
        in_specs=[
            pl.BlockSpec((TB, H), lambda i: (i, 0)),
            pl.BlockSpec((NC, TB, H), lambda i: (0, i, 0)),
            pl.BlockSpec((H, 3 * H), lambda i: (0, 0)),
            pl.BlockSpec((H, 3 * H), lambda i: (0, 0)),
            pl.BlockSpec((1, 3 * H), lambda i: (0, 0)),
            pl.BlockSpec((1, 3 * H), lambda i: (0, 0)),
            pl.BlockSpec(memory_space=pl.ANY),
        ],
        out_specs=pl.BlockSpec((TB, H), lambda i: (i, 0)),
        out_shape=jax.ShapeDtypeStruct((E, H), jnp.float32),
        input_output_aliases={6: 0},
    )(edge_features, part, wih, whh, bih, bhh, out)
    return out


# speedup vs baseline: 5.8571x; 1.0281x over previous
import functools

import jax
import jax.numpy as jnp
from jax import lax
from jax.experimental import pallas as pl
from jax.experimental.pallas import tpu as pltpu
from jax.experimental.pallas import tpu_sc as plsc

H = 128
N_NODES = 10000
E_TOTAL = 320000

NC = 2
NS = 16
NW = NC * NS

SLABS = (140800, 102400, 76800)
CH = 128

ZR = 632
N_ACC = ZR * NS
ZFULL = ZR // CH
ZTAIL = ZR - ZFULL * CH

TA = 12800
TB = 2000
NT = N_NODES // TB


def _make_scatter_body(epw, nch, seeded):
    def _scatter_body(msg_hbm, idx_hbm, init_hbm, out_hbm,
                      idx_v, buf0, buf1, acc, sem0, sem1, semi):
        c = lax.axis_index("c")
        s = lax.axis_index("s")
        wid = c * NS + s
        base = wid * epw

        pltpu.async_copy(idx_hbm.at[wid], idx_v, semi)
        if seeded:
            pltpu.sync_copy(init_hbm.at[c, pl.ds(s * ZR, ZR)],
                            acc.at[pl.ds(s * ZR, ZR)])
        else:
            pltpu.sync_copy(init_hbm, buf0)
            for t in range(ZFULL):
                pltpu.sync_copy(buf0, acc.at[pl.ds(s * ZR + t * CH, CH)])
            pltpu.sync_copy(buf0.at[pl.ds(0, ZTAIL)],
                            acc.at[pl.ds(s * ZR + ZFULL * CH, ZTAIL)])
        pltpu.async_copy(msg_hbm.at[pl.ds(base, CH)], buf0, sem0)
        pltpu.make_async_copy(idx_hbm.at[wid], idx_v, semi).wait()
        plsc.subcore_barrier()

        def step(k, carry):
            j0 = 2 * k
            pltpu.async_copy(msg_hbm.at[pl.ds(base + (j0 + 1) * CH, CH)], buf1, sem1)
            pltpu.make_async_copy(msg_hbm.at[pl.ds(base + j0 * CH, CH)], buf0, sem0).wait()
            pltpu.sync_copy(buf0, acc.at[idx_v.at[j0]], add=True)

            @pl.when(j0 + 2 < nch)
            def _():
                pltpu.async_copy(msg_hbm.at[pl.ds(base + (j0 + 2) * CH, CH)], buf0, sem0)

            pltpu.make_async_copy(msg_hbm.at[pl.ds(base + (j0 + 1) * CH, CH)], buf1, sem1).wait()
            pltpu.sync_copy(buf1, acc.at[idx_v.at[j0 + 1]], add=True)
            return carry

        lax.fori_loop(0, nch // 2, step, 0)
        plsc.subcore_barrier()
        pltpu.sync_copy(acc.at[pl.ds(s * ZR, ZR)], out_hbm.at[c, pl.ds(s * ZR, ZR)])

    return _scatter_body


def _sc_scatter(message, idx3, init, epw, nch, seeded):
    mesh = plsc.VectorSubcoreMesh(core_axis_name="c", subcore_axis_name="s")
    scatter = pl.kernel(
        _make_scatter_body(epw, nch, seeded),
        out_type=jax.ShapeDtypeStruct((NC, N_ACC, H), jnp.float32),
        mesh=mesh,
        scratch_types=[
            pltpu.VMEM((nch, CH), jnp.int32),
            pltpu.VMEM((CH, H), jnp.float32),
            pltpu.VMEM((CH, H), jnp.float32),
            pltpu.VMEM_SHARED((N_ACC, H), jnp.float32),
            pltpu.SemaphoreType.DMA,
            pltpu.SemaphoreType.DMA,
            pltpu.SemaphoreType.DMA,
        ],
    )
    return scatter(message, idx3, init)


def _pad_idx(dest_slice, epw, nch):
    d2 = dest_slice.reshape(NW, epw)
    d2 = jnp.pad(d2, ((0, 0), (0, nch * CH - epw)), constant_values=N_NODES)
    return d2.reshape(NW, nch, CH)


def _fused_body(ef_ref, wm_ref, bm_ref, whh_ref, bih_ref, bhh_ref,
                msg_ref, out_ref):
    h = ef_ref[...]
    m = jnp.dot(h, wm_ref[...], preferred_element_type=jnp.float32) + bm_ref[...]
    msg_ref[...] = jnp.maximum(m, 0.0)
    gh = jnp.dot(h, whh_ref[...], preferred_element_type=jnp.float32) + bhh_ref[...]
    gi = bih_ref[...]
    r = jax.nn.sigmoid(gi[:, :H] + gh[:, :H])
    z = jax.nn.sigmoid(gi[:, H:2 * H] + gh[:, H:2 * H])
    n = jnp.tanh(gi[:, 2 * H:] + r * gh[:, 2 * H:])
    out_ref[...] = (1.0 - z) * n + z * h


def _fused_donor_body(ef_ref, wm_ref, bm_ref, whh_ref, bih_ref, bhh_ref,
                      big_ref, msg_ref, out_ref):
    del big_ref
    _fused_body(ef_ref, wm_ref, bm_ref, whh_ref, bih_ref, bhh_ref,
                msg_ref, out_ref)


def _gru_small_body(ef_ref, p_ref, wih_ref, whh_ref,
                    bih_ref, bhh_ref, big_ref, out_ref):
    del big_ref
    h = ef_ref[...]
    gh = jnp.dot(h, whh_ref[...], preferred_element_type=jnp.float32) + bhh_ref[...]
    x = p_ref[0] + p_ref[1]
    gi = jnp.dot(x, wih_ref[...], preferred_element_type=jnp.float32) + bih_ref[...]
    r = jax.nn.sigmoid(gi[:, :H] + gh[:, :H])
    z = jax.nn.sigmoid(gi[:, H:2 * H] + gh[:, H:2 * H])
    n = jnp.tanh(gi[:, 2 * H:] + r * gh[:, 2 * H:])
    out_ref[...] = (1.0 - z) * n + z * h


def kernel(edge_features, edge_index, W_msg_w, W_msg_b, W_ih, W_hh, b_ih, b_hh):
    E = edge_features.shape[0]
    wm = W_msg_w.T
    bm = W_msg_b.reshape(1, H)
    wih = W_ih.T
    whh = W_hh.T
    bih = b_ih.reshape(1, 3 * H)
    bhh = b_hh.reshape(1, 3 * H)

    dest = edge_index[1]
    zeros = jnp.zeros((CH, H), dtype=jnp.float32)

    w_specs = [
        pl.BlockSpec((H, H), lambda i: (0, 0)),
        pl.BlockSpec((1, H), lambda i: (0, 0)),
        pl.BlockSpec((H, 3 * H), lambda i: (0, 0)),
        pl.BlockSpec((1, 3 * H), lambda i: (0, 0)),
        pl.BlockSpec((1, 3 * H), lambda i: (0, 0)),
    ]

    out = None
    part = zeros
    seeded = False
    start = 0
    for slab in SLABS:
        epw = slab // NW
        nch = -(-epw // CH)
        if nch % 2:
            nch += 1
        offs = start // TA

        in_specs = [pl.BlockSpec((TA, H),
                                 functools.partial(lambda o, i: (i + o, 0), offs))]
        in_specs += w_specs
        args = [edge_features, wm, bm, whh, bih, bhh]
        body = _fused_body
        aliases = {}
        if out is not None:
            in_specs += [pl.BlockSpec(memory_space=pl.ANY)]
            args += [out]
            body = _fused_donor_body
            aliases = {6: 1}

        msg, out = pl.pallas_call(
            body,
            grid=(slab // TA,),
            in_specs=in_specs,
            out_specs=[
                pl.BlockSpec((TA, H), lambda i: (i, 0)),
                pl.BlockSpec((TA, H),
                             functools.partial(lambda o, i: (i + o, 0), offs)),
            ],
            out_shape=[
                jax.ShapeDtypeStruct((slab + nch * CH - epw, H), jnp.float32),
                jax.ShapeDtypeStruct((E, H), jnp.float32),
            ],
            input_output_aliases=aliases,
        )(*args)

        idx3 = _pad_idx(dest[start:start + slab], epw, nch)
        part = _sc_scatter(msg, idx3, part, epw, nch, seeded)
        seeded = True
        start += slab

    out = pl.pallas_call(
        _gru_small_body,
        grid=(NT,),
        in_specs=[
            pl.BlockSpec((TB, H), lambda i: (i, 0)),
            pl.BlockSpec((NC, TB, H), lambda i: (0, i, 0)),
            pl.BlockSpec((H, 3 * H), lambda i: (0, 0)),
            pl.BlockSpec((H, 3 * H), lambda i: (0, 0)),
            pl.BlockSpec((1, 3 * H), lambda i: (0, 0)),
            pl.BlockSpec((1, 3 * H), lambda i: (0, 0)),
            pl.BlockSpec(memory_space=pl.ANY),
        ],
        out_specs=pl.BlockSpec((TB, H), lambda i: (i, 0)),
        out_shape=jax.ShapeDtypeStruct((E, H), jnp.float32),
        input_output_aliases={6: 0},
    )(edge_features, part, wih, whh, bih, bhh, out)
    return out
